# Initial kernel scaffold; baseline (speedup 1.0000x reference)
import functools

import jax
import jax.numpy as jnp
from jax import lax
from jax.experimental import pallas as pl
from jax.experimental.pallas import tpu as pltpu
from jax.experimental.pallas import tpu_sc as plsc

D = 128
NC = 2
NS = 16
NW = NC * NS

CHUNK = 256


def _make_gather(n_flat):
    b_per_w = n_flat // NW
    n_chunks = b_per_w // CHUNK
    mesh = plsc.VectorSubcoreMesh(core_axis_name="c", subcore_axis_name="s")

    @functools.partial(
        pl.kernel,
        mesh=mesh,
        out_type=jax.ShapeDtypeStruct((n_flat, D), jnp.float32),
        scratch_types=[
            pltpu.VMEM((n_chunks, CHUNK), jnp.int32),
            pltpu.VMEM((2, CHUNK, D), jnp.float32),
            pltpu.SemaphoreType.DMA,
            pltpu.SemaphoreType.DMA,
        ],
    )
    def gather_kernel(idx_hbm, table_hbm, out_hbm, idx_v, rows_v, g_sem, s_sem):
        wid = lax.axis_index("s") * NC + lax.axis_index("c")
        base = wid * b_per_w
        pltpu.sync_copy(idx_hbm.at[wid], idx_v)

        gathers = [pltpu.async_copy(table_hbm.at[idx_v.at[0]], rows_v.at[0], g_sem)]
        stores = []
        for c in range(n_chunks):
            gathers[c].wait()
            if c >= 1:
                stores[c - 1].wait()
            if c + 1 < n_chunks:
                gathers.append(
                    pltpu.async_copy(
                        table_hbm.at[idx_v.at[c + 1]], rows_v.at[(c + 1) % 2], g_sem
                    )
                )
            stores.append(
                pltpu.async_copy(
                    rows_v.at[c % 2], out_hbm.at[pl.ds(base + c * CHUNK, CHUNK)], s_sem
                )
            )
        stores[n_chunks - 1].wait()

    return gather_kernel


def kernel(input_ids, token_embedding):
    b, l = input_ids.shape
    n_flat = b * l
    idx = input_ids.reshape(NW, (n_flat // NW) // CHUNK, CHUNK).astype(jnp.int32)
    out = _make_gather(n_flat)(idx, token_embedding)
    return out.reshape(b, l, D)


# speedup vs baseline: 1.3831x; 1.3831x over previous
import functools

import jax
import jax.numpy as jnp
from jax import lax
from jax.experimental import pallas as pl
from jax.experimental.pallas import tpu as pltpu
from jax.experimental.pallas import tpu_sc as plsc

D = 128
NC = 2
NS = 16
NW = NC * NS

CHUNK = 128


def _make_gather(n_flat):
    b_per_w = n_flat // NW
    n_chunks = b_per_w // CHUNK
    mesh = plsc.VectorSubcoreMesh(core_axis_name="c", subcore_axis_name="s")

    @functools.partial(
        pl.kernel,
        mesh=mesh,
        out_type=jax.ShapeDtypeStruct((n_flat, D), jnp.float32),
        scratch_types=[
            pltpu.VMEM((n_chunks, CHUNK), jnp.int32),
            pltpu.VMEM((2, CHUNK, D), jnp.float32),
            pltpu.SemaphoreType.DMA,
            pltpu.SemaphoreType.DMA,
        ],
    )
    def gather_kernel(idx_hbm, table_hbm, out_hbm, idx_v, rows_v, g_sem, s_sem):
        wid = lax.axis_index("s") * NC + lax.axis_index("c")
        base = wid * b_per_w
        pltpu.sync_copy(idx_hbm.at[wid], idx_v)

        gathers = [pltpu.async_copy(table_hbm.at[idx_v.at[0]], rows_v.at[0], g_sem)]
        stores = []
        for c in range(n_chunks):
            gathers[c].wait()
            if c >= 1:
                stores[c - 1].wait()
            if c + 1 < n_chunks:
                gathers.append(
                    pltpu.async_copy(
                        table_hbm.at[idx_v.at[c + 1]], rows_v.at[(c + 1) % 2], g_sem
                    )
                )
            stores.append(
                pltpu.async_copy(
                    rows_v.at[c % 2], out_hbm.at[pl.ds(base + c * CHUNK, CHUNK)], s_sem
                )
            )
        stores[n_chunks - 1].wait()

    return gather_kernel


def kernel(input_ids, token_embedding):
    b, l = input_ids.shape
    n_flat = b * l
    idx = input_ids.reshape(NW, (n_flat // NW) // CHUNK, CHUNK).astype(jnp.int32)
    out = _make_gather(n_flat)(idx, token_embedding)
    return out.reshape(b, l, D)


# speedup vs baseline: 1.5334x; 1.1087x over previous
import functools

import jax
import jax.numpy as jnp
from jax import lax
from jax.experimental import pallas as pl
from jax.experimental.pallas import tpu as pltpu
from jax.experimental.pallas import tpu_sc as plsc

D = 128
NC = 2
NS = 16
NW = NC * NS

CHUNK = 128
NBUF = 4


def _make_gather(n_flat):
    b_per_w = n_flat // NW
    n_chunks = b_per_w // CHUNK
    mesh = plsc.VectorSubcoreMesh(core_axis_name="c", subcore_axis_name="s")

    @functools.partial(
        pl.kernel,
        mesh=mesh,
        out_type=jax.ShapeDtypeStruct((n_flat, D), jnp.float32),
        scratch_types=[
            pltpu.VMEM((n_chunks, CHUNK), jnp.int32),
            pltpu.VMEM((NBUF, CHUNK, D), jnp.float32),
            pltpu.SemaphoreType.DMA,
            pltpu.SemaphoreType.DMA,
        ],
    )
    def gather_kernel(idx_hbm, table_hbm, out_hbm, idx_v, rows_v, g_sem, s_sem):
        wid = lax.axis_index("s") * NC + lax.axis_index("c")
        base = wid * b_per_w
        pltpu.sync_copy(idx_hbm.at[wid], idx_v)

        gathers = [
            pltpu.async_copy(table_hbm.at[idx_v.at[g]], rows_v.at[g % NBUF], g_sem)
            for g in range(min(NBUF - 1, n_chunks))
        ]
        stores = []
        for c in range(n_chunks):
            gathers[c].wait()
            if c >= 1:
                stores[c - 1].wait()
            g = c + NBUF - 1
            if g < n_chunks:
                gathers.append(
                    pltpu.async_copy(
                        table_hbm.at[idx_v.at[g]], rows_v.at[g % NBUF], g_sem
                    )
                )
            stores.append(
                pltpu.async_copy(
                    rows_v.at[c % NBUF], out_hbm.at[pl.ds(base + c * CHUNK, CHUNK)], s_sem
                )
            )
        stores[n_chunks - 1].wait()

    return gather_kernel


def kernel(input_ids, token_embedding):
    b, l = input_ids.shape
    n_flat = b * l
    idx = input_ids.reshape(NW, (n_flat // NW) // CHUNK, CHUNK).astype(jnp.int32)
    out = _make_gather(n_flat)(idx, token_embedding)
    return out.reshape(b, l, D)
